# Initial kernel scaffold; baseline (speedup 1.0000x reference)
import jax
import jax.numpy as jnp
from jax.experimental import pallas as pl


def kernel(hidden_states, W_base, W_gate, lora_A, lora_B):
    raise NotImplementedError("write your pallas kernel here")


# speedup vs baseline: 5.4284x; 5.4284x over previous
import functools

import jax
import jax.numpy as jnp
from jax.experimental import pallas as pl

T = 2048
D = 2048
E = 8
R = 32
SCALING = 64 / 32
ER = E * R

TILE_T = 256


def _fused_kernel(x_ref, wb_ref, wg_ref, a2_ref, b2_ref, o_ref):
    x = x_ref[...]
    logits = jnp.dot(x, wg_ref[...], preferred_element_type=jnp.float32)
    cols = jax.lax.broadcasted_iota(jnp.int32, logits.shape, 1)
    m1 = jnp.max(logits, axis=1, keepdims=True)
    i1 = jnp.min(jnp.where(logits == m1, cols, E), axis=1, keepdims=True)
    masked = jnp.where(cols == i1, -jnp.inf, logits)
    m2 = jnp.max(masked, axis=1, keepdims=True)
    i2 = jnp.min(jnp.where(masked == m2, cols, E), axis=1, keepdims=True)
    e2 = jnp.exp(m2 - m1)
    denom = 1.0 + e2
    w1 = 1.0 / denom
    w2 = e2 / denom

    a = jnp.dot(x, a2_ref[...], preferred_element_type=jnp.float32)
    ecol = jax.lax.broadcasted_iota(jnp.int32, a.shape, 1) // R
    w_cols = jnp.where(ecol == i1, w1, 0.0) + jnp.where(ecol == i2, w2, 0.0)
    moe = jnp.dot(a * w_cols, b2_ref[...], preferred_element_type=jnp.float32)

    base = jnp.dot(x, wb_ref[...], preferred_element_type=jnp.float32)
    o_ref[...] = base + moe * SCALING


@jax.jit
def kernel(hidden_states, W_base, W_gate, lora_A, lora_B):
    A_all = lora_A.reshape(ER, D).T
    B_all = lora_B.transpose(0, 2, 1).reshape(ER, D)

    grid = (T // TILE_T,)
    return pl.pallas_call(
        _fused_kernel,
        grid=grid,
        in_specs=[
            pl.BlockSpec((TILE_T, D), lambda i: (i, 0)),
            pl.BlockSpec((D, D), lambda i: (0, 0)),
            pl.BlockSpec((D, E), lambda i: (0, 0)),
            pl.BlockSpec((D, ER), lambda i: (0, 0)),
            pl.BlockSpec((ER, D), lambda i: (0, 0)),
        ],
        out_specs=pl.BlockSpec((TILE_T, D), lambda i: (i, 0)),
        out_shape=jax.ShapeDtypeStruct((T, D), jnp.float32),
    )(hidden_states, W_base, W_gate, A_all, B_all)
